# baseline (speedup 1.0000x reference)
import functools

import jax
import jax.numpy as jnp
from jax.experimental import pallas as pl
from jax.experimental.pallas import tpu as pltpu


_DN = (((1,), (0,)), ((), ()))


def _hop_first(adj_ref, xbf_ref, xf_ref, w0t_ref, w1t_ref, b_ref,
               part_ref, ybf_ref, adjbf_ref):
    a = adj_ref[...]
    ab = a.astype(jnp.bfloat16)
    adjbf_ref[...] = ab
    y = jax.lax.dot_general(ab, xbf_ref[...], _DN,
                            preferred_element_type=jnp.float32)
    ybf_ref[...] = y.astype(jnp.bfloat16)
    part_ref[...] = (
        b_ref[...]
        + jnp.dot(xf_ref[...], w0t_ref[...], preferred_element_type=jnp.float32)
        + jnp.dot(y, w1t_ref[...], preferred_element_type=jnp.float32)
    )


def _hops_rest(bm, nb,
               adjbf_ref, y1bf_ref, part_in_ref, wts_ref,
               out_ref, ybuf_ref, acc_ref, sem):
    h = pl.program_id(0)
    m = pl.program_id(1)
    nh = pl.num_programs(0)
    rows = pl.ds(m * bm, bm)
    src = jax.lax.rem(h, 2)

    @pl.when((h == 0) & (m == 0))
    def _():
        ybuf_ref[0] = y1bf_ref[...]

    y = jax.lax.dot_general(adjbf_ref[...], ybuf_ref[src],
                            _DN, preferred_element_type=jnp.float32)
    part = jnp.dot(y, wts_ref[h], preferred_element_type=jnp.float32)

    @pl.when(h == 0)
    def _():
        acc_ref[rows, :] = part_in_ref[...] + part

    @pl.when(h > 0)
    def _():
        acc_ref[rows, :] = acc_ref[rows, :] + part

    @pl.when(h < nh - 1)
    def _():
        ybuf_ref[1 - src, rows, :] = y.astype(jnp.bfloat16)

    @pl.when(h == nh - 1)
    def _():
        pltpu.make_async_copy(acc_ref.at[rows, :], out_ref.at[rows, :],
                              sem).start()

    @pl.when((h == nh - 1) & (m == nb - 1))
    def _():
        for i in range(nb):
            blk = pl.ds(i * bm, bm)
            pltpu.make_async_copy(acc_ref.at[blk, :], out_ref.at[blk, :],
                                  sem).wait()


@jax.jit
def kernel(x, adj, W, b):
    N, D = x.shape
    DO = W.shape[0]
    K = W.shape[1] // D - 1

    xbf = x.astype(jnp.bfloat16)
    WT = W.T.astype(jnp.float32)
    wts = [WT[k * D:(k + 1) * D] for k in range(K + 1)]
    wrest = jnp.stack(wts[2:])
    b2 = b.reshape(1, DO).astype(jnp.float32)

    BM = 400 if N % 400 == 0 else 16
    NB = N // BM
    params = pltpu.CompilerParams(dimension_semantics=("arbitrary",))
    row_blk = lambda i: (i, 0)
    full_blk = lambda i: (0, 0)

    part1, y1bf, adjbf = pl.pallas_call(
        _hop_first,
        grid=(NB,),
        in_specs=[
            pl.BlockSpec((BM, N), row_blk),
            pl.BlockSpec((N, D), full_blk),
            pl.BlockSpec((BM, D), row_blk),
            pl.BlockSpec((D, DO), full_blk),
            pl.BlockSpec((D, DO), full_blk),
            pl.BlockSpec((1, DO), full_blk),
        ],
        out_specs=[
            pl.BlockSpec((BM, DO), row_blk),
            pl.BlockSpec((BM, D), row_blk),
            pl.BlockSpec((BM, N), row_blk),
        ],
        out_shape=[
            jax.ShapeDtypeStruct((N, DO), jnp.float32),
            jax.ShapeDtypeStruct((N, D), jnp.bfloat16),
            jax.ShapeDtypeStruct((N, N), jnp.bfloat16),
        ],
        compiler_params=params,
    )(adj, xbf, x, wts[0], wts[1], b2)

    out = pl.pallas_call(
        functools.partial(_hops_rest, BM, NB),
        grid=(K - 1, NB),
        in_specs=[
            pl.BlockSpec((BM, N), lambda h, m: (m, 0)),
            pl.BlockSpec((N, D), lambda h, m: (0, 0)),
            pl.BlockSpec((BM, DO), lambda h, m: (m * (h == 0), 0)),
            pl.BlockSpec((K - 1, D, DO), lambda h, m: (0, 0, 0)),
        ],
        out_specs=pl.BlockSpec(memory_space=pl.ANY),
        out_shape=jax.ShapeDtypeStruct((N, DO), jnp.float32),
        scratch_shapes=[
            pltpu.VMEM((2, N, D), jnp.bfloat16),
            pltpu.VMEM((N, DO), jnp.float32),
            pltpu.SemaphoreType.DMA,
        ],
        compiler_params=pltpu.CompilerParams(
            dimension_semantics=("arbitrary", "arbitrary")),
    )(adjbf, y1bf, part1, wrest)
    return out


# speedup vs baseline: 1.0331x; 1.0331x over previous
import functools

import jax
import jax.numpy as jnp
from jax.experimental import pallas as pl
from jax.experimental.pallas import tpu as pltpu


_DN = (((1,), (0,)), ((), ()))


def _hop_first(bm, adj_ref, xf_ref, w0t_ref, w1t_ref, b_ref,
               part_ref, ybf_ref, adjbf_ref, xbf_ref):
    m = pl.program_id(0)
    rows = pl.ds(m * bm, bm)

    @pl.when(m == 0)
    def _():
        xbf_ref[...] = xf_ref[...].astype(jnp.bfloat16)

    a = adj_ref[...]
    ab = a.astype(jnp.bfloat16)
    adjbf_ref[...] = ab
    y = jax.lax.dot_general(ab, xbf_ref[...], _DN,
                            preferred_element_type=jnp.float32)
    ybf_ref[...] = y.astype(jnp.bfloat16)
    part_ref[...] = (
        b_ref[...]
        + jnp.dot(xf_ref[rows, :], w0t_ref[...],
                  preferred_element_type=jnp.float32)
        + jnp.dot(y, w1t_ref[...], preferred_element_type=jnp.float32)
    )


def _hops_rest(bm, nb,
               adjbf_ref, y1bf_ref, part_in_ref, wts_ref,
               out_ref, ybuf_ref, acc_ref, sem):
    h = pl.program_id(0)
    m = pl.program_id(1)
    nh = pl.num_programs(0)
    rows = pl.ds(m * bm, bm)
    src = jax.lax.rem(h, 2)

    @pl.when((h == 0) & (m == 0))
    def _():
        ybuf_ref[0] = y1bf_ref[...]

    y = jax.lax.dot_general(adjbf_ref[...], ybuf_ref[src],
                            _DN, preferred_element_type=jnp.float32)
    part = jnp.dot(y, wts_ref[h], preferred_element_type=jnp.float32)

    @pl.when(h == 0)
    def _():
        acc_ref[rows, :] = part_in_ref[...] + part

    @pl.when(h > 0)
    def _():
        acc_ref[rows, :] = acc_ref[rows, :] + part

    @pl.when(h < nh - 1)
    def _():
        ybuf_ref[1 - src, rows, :] = y.astype(jnp.bfloat16)

    @pl.when(h == nh - 1)
    def _():
        pltpu.make_async_copy(acc_ref.at[rows, :], out_ref.at[rows, :],
                              sem).start()

    @pl.when((h == nh - 1) & (m == nb - 1))
    def _():
        for i in range(nb):
            blk = pl.ds(i * bm, bm)
            pltpu.make_async_copy(acc_ref.at[blk, :], out_ref.at[blk, :],
                                  sem).wait()


@jax.jit
def kernel(x, adj, W, b):
    N, D = x.shape
    DO = W.shape[0]
    K = W.shape[1] // D - 1

    WT = W.T.astype(jnp.float32)
    wts = [WT[k * D:(k + 1) * D] for k in range(K + 1)]
    wrest = jnp.stack(wts[2:])
    b2 = b.reshape(1, DO).astype(jnp.float32)

    BM = 400 if N % 400 == 0 else 16
    NB = N // BM
    params = pltpu.CompilerParams(dimension_semantics=("arbitrary",))
    row_blk = lambda i: (i, 0)
    full_blk = lambda i: (0, 0)

    part1, y1bf, adjbf = pl.pallas_call(
        functools.partial(_hop_first, BM),
        grid=(NB,),
        in_specs=[
            pl.BlockSpec((BM, N), row_blk),
            pl.BlockSpec((N, D), full_blk),
            pl.BlockSpec((D, DO), full_blk),
            pl.BlockSpec((D, DO), full_blk),
            pl.BlockSpec((1, DO), full_blk),
        ],
        out_specs=[
            pl.BlockSpec((BM, DO), row_blk),
            pl.BlockSpec((BM, D), row_blk),
            pl.BlockSpec((BM, N), row_blk),
        ],
        out_shape=[
            jax.ShapeDtypeStruct((N, DO), jnp.float32),
            jax.ShapeDtypeStruct((N, D), jnp.bfloat16),
            jax.ShapeDtypeStruct((N, N), jnp.bfloat16),
        ],
        scratch_shapes=[pltpu.VMEM((N, D), jnp.bfloat16)],
        compiler_params=params,
    )(adj, x, wts[0], wts[1], b2)

    out = pl.pallas_call(
        functools.partial(_hops_rest, BM, NB),
        grid=(K - 1, NB),
        in_specs=[
            pl.BlockSpec((BM, N), lambda h, m: (m, 0)),
            pl.BlockSpec((N, D), lambda h, m: (0, 0)),
            pl.BlockSpec((BM, DO), lambda h, m: (m * (h == 0), 0)),
            pl.BlockSpec((K - 1, D, DO), lambda h, m: (0, 0, 0)),
        ],
        out_specs=pl.BlockSpec(memory_space=pl.ANY),
        out_shape=jax.ShapeDtypeStruct((N, DO), jnp.float32),
        scratch_shapes=[
            pltpu.VMEM((2, N, D), jnp.bfloat16),
            pltpu.VMEM((N, DO), jnp.float32),
            pltpu.SemaphoreType.DMA,
        ],
        compiler_params=pltpu.CompilerParams(
            dimension_semantics=("arbitrary", "arbitrary")),
    )(adjbf, y1bf, part1, wrest)
    return out


# speedup vs baseline: 1.0888x; 1.0540x over previous
import functools

import jax
import jax.numpy as jnp
from jax.experimental import pallas as pl
from jax.experimental.pallas import tpu as pltpu


_DN = (((1,), (0,)), ((), ()))


def _hop_first(bm, adj_ref, xf_ref, w0t_ref, w1t_ref, b_ref,
               part_ref, ybf_ref, adjbf_ref, xbf_ref):
    m = pl.program_id(0)
    rows = pl.ds(m * bm, bm)

    @pl.when(m == 0)
    def _():
        xbf_ref[...] = xf_ref[...].astype(jnp.bfloat16)

    a = adj_ref[...]
    ab = a.astype(jnp.bfloat16)
    adjbf_ref[...] = ab
    y = jax.lax.dot_general(ab, xbf_ref[...], _DN,
                            preferred_element_type=jnp.float32)
    ybf_ref[...] = y.astype(jnp.bfloat16)
    part_ref[...] = (
        b_ref[...]
        + jnp.dot(xf_ref[rows, :], w0t_ref[...],
                  preferred_element_type=jnp.float32)
        + jnp.dot(y, w1t_ref[...], preferred_element_type=jnp.float32)
    )


def _hops_rest(n, bmr, bk,
               adjbf_ref, y1bf_ref, part_in_ref, wts_ref,
               out_ref, ybuf_ref, acc_ref, yacc_ref, sem):
    h = pl.program_id(0)
    m = pl.program_id(1)
    k = pl.program_id(2)
    nh = pl.num_programs(0)
    nm = pl.num_programs(1)
    nk = pl.num_programs(2)
    rows = pl.ds(m * bmr, bmr)
    src = jax.lax.rem(h, 2)

    npad = ybuf_ref.shape[1]

    @pl.when((h == 0) & (m == 0) & (k == 0))
    def _():
        ybuf_ref[0, pl.ds(0, n), :] = y1bf_ref[...]
        if npad > n:
            ybuf_ref[0, pl.ds(n, npad - n), :] = jnp.zeros(
                (npad - n, ybuf_ref.shape[2]), jnp.bfloat16)
            ybuf_ref[1, pl.ds(n, npad - n), :] = jnp.zeros(
                (npad - n, ybuf_ref.shape[2]), jnp.bfloat16)

    a = adjbf_ref[...]
    col = jax.lax.broadcasted_iota(jnp.int32, a.shape, 1)
    a = jnp.where(col < n - k * bk, a, jnp.bfloat16(0))
    partial = jax.lax.dot_general(
        a, ybuf_ref[src, pl.ds(k * bk, bk), :],
        _DN, preferred_element_type=jnp.float32)

    @pl.when(k == 0)
    def _():
        yacc_ref[...] = partial

    @pl.when(k > 0)
    def _():
        yacc_ref[...] = yacc_ref[...] + partial

    @pl.when(k == nk - 1)
    def _():
        y = yacc_ref[...]
        part = jnp.dot(y, wts_ref[h], preferred_element_type=jnp.float32)

        @pl.when(h == 0)
        def _():
            acc_ref[rows, :] = part_in_ref[...] + part

        @pl.when(h > 0)
        def _():
            acc_ref[rows, :] = acc_ref[rows, :] + part

        @pl.when(h < nh - 1)
        def _():
            row = jax.lax.broadcasted_iota(jnp.int32, y.shape, 0)
            yst = jnp.where(row < n - m * bmr, y.astype(jnp.bfloat16),
                            jnp.bfloat16(0))
            ybuf_ref[1 - src, rows, :] = yst

        @pl.when(h == nh - 1)
        def _():
            last = n - (nm - 1) * bmr

            @pl.when(m < nm - 1)
            def _():
                pltpu.make_async_copy(
                    acc_ref.at[rows, :], out_ref.at[rows, :], sem).start()

            @pl.when(m == nm - 1)
            def _():
                lrows = pl.ds((nm - 1) * bmr, last)
                pltpu.make_async_copy(
                    acc_ref.at[lrows, :], out_ref.at[lrows, :], sem).start()
                for i in range(nm - 1):
                    blk = pl.ds(i * bmr, bmr)
                    pltpu.make_async_copy(
                        acc_ref.at[blk, :], out_ref.at[blk, :], sem).wait()
                pltpu.make_async_copy(
                    acc_ref.at[lrows, :], out_ref.at[lrows, :], sem).wait()


@jax.jit
def kernel(x, adj, W, b):
    N, D = x.shape
    DO = W.shape[0]
    K = W.shape[1] // D - 1

    WT = W.T.astype(jnp.float32)
    wts = [WT[k * D:(k + 1) * D] for k in range(K + 1)]
    wrest = jnp.stack(wts[2:])
    b2 = b.reshape(1, DO).astype(jnp.float32)

    BM = 400 if N % 400 == 0 else 16
    NB = N // BM
    params = pltpu.CompilerParams(dimension_semantics=("arbitrary",))
    row_blk = lambda i: (i, 0)
    full_blk = lambda i: (0, 0)

    part1, y1bf, adjbf = pl.pallas_call(
        functools.partial(_hop_first, BM),
        grid=(NB,),
        in_specs=[
            pl.BlockSpec((BM, N), row_blk),
            pl.BlockSpec((N, D), full_blk),
            pl.BlockSpec((D, DO), full_blk),
            pl.BlockSpec((D, DO), full_blk),
            pl.BlockSpec((1, DO), full_blk),
        ],
        out_specs=[
            pl.BlockSpec((BM, DO), row_blk),
            pl.BlockSpec((BM, D), row_blk),
            pl.BlockSpec((BM, N), row_blk),
        ],
        out_shape=[
            jax.ShapeDtypeStruct((N, DO), jnp.float32),
            jax.ShapeDtypeStruct((N, D), jnp.bfloat16),
            jax.ShapeDtypeStruct((N, N), jnp.bfloat16),
        ],
        scratch_shapes=[pltpu.VMEM((N, D), jnp.bfloat16)],
        compiler_params=params,
    )(adj, x, wts[0], wts[1], b2)

    BMR = 2048
    NM = -(-N // BMR)
    BK = 5120
    NK = -(-N // BK)
    NPAD = max(NM * BMR, NK * BK)

    out = pl.pallas_call(
        functools.partial(_hops_rest, N, BMR, BK),
        grid=(K - 1, NM, NK),
        in_specs=[
            pl.BlockSpec((BMR, BK), lambda h, m, k: (m, k)),
            pl.BlockSpec((N, D), lambda h, m, k: (0, 0)),
            pl.BlockSpec((BMR, DO), lambda h, m, k: (m * (h == 0), 0)),
            pl.BlockSpec((K - 1, D, DO), lambda h, m, k: (0, 0, 0)),
        ],
        out_specs=pl.BlockSpec(memory_space=pl.ANY),
        out_shape=jax.ShapeDtypeStruct((N, DO), jnp.float32),
        scratch_shapes=[
            pltpu.VMEM((2, NPAD, D), jnp.bfloat16),
            pltpu.VMEM((NPAD, DO), jnp.float32),
            pltpu.VMEM((BMR, DO), jnp.float32),
            pltpu.SemaphoreType.DMA,
        ],
        compiler_params=pltpu.CompilerParams(
            dimension_semantics=("arbitrary", "arbitrary", "arbitrary"),
            vmem_limit_bytes=64 * 1024 * 1024),
    )(adjbf, y1bf, part1, wrest)
    return out
